# baseline (speedup 1.0000x reference)
import functools

import jax
import jax.numpy as jnp
from jax import lax
from jax.experimental import pallas as pl
from jax.experimental.pallas import tpu as pltpu
from jax.experimental.pallas import tpu_sc as plsc

B = 16384
F = 26
V = 100000
D = 16

NC = 2
NSC = 16
NW = NC * NSC
BPW = B // NW
CB = 128
NCHUNK = BPW // CB

_sc_mesh = plsc.VectorSubcoreMesh(core_axis_name="c", subcore_axis_name="s")


@functools.partial(
    pl.kernel,
    mesh=_sc_mesh,
    compiler_params=pltpu.CompilerParams(use_tc_tiling_on_sc=False),
    out_type=jax.ShapeDtypeStruct((B, D), jnp.float32),
    scratch_types=[
        pltpu.VMEM((2, CB * F), jnp.int32),
        pltpu.VMEM((2, CB * F, D), jnp.float32),
        pltpu.VMEM((CB, D), jnp.float32),
        pltpu.SemaphoreType.DMA,
        pltpu.SemaphoreType.DMA,
    ],
)
def _pool(tflat, fidx, out, idx_v, rows_v, out_v, sem0, sem1):
    wid = lax.axis_index("s") * NC + lax.axis_index("c")
    base = wid * BPW
    sems = (sem0, sem1)

    def fire(c):
        b = c % 2
        row0 = base + c * CB
        pltpu.sync_copy(fidx.at[pl.ds(row0 * F, CB * F)], idx_v.at[b])
        return pltpu.async_copy(tflat.at[idx_v.at[b]], rows_v.at[b], sems[b])

    inflight = fire(0)
    for c in range(NCHUNK):
        b = c % 2
        row0 = base + c * CB
        cp = inflight
        if c + 1 < NCHUNK:
            inflight = fire(c + 1)
        cp.wait()

        def body(bb, _):
            acc = rows_v[b, bb * F, :]
            for f in range(1, F):
                acc = acc + rows_v[b, bb * F + f, :]
            out_v[bb, :] = acc
            return 0

        lax.fori_loop(0, CB, body, 0)
        pltpu.sync_copy(out_v, out.at[pl.ds(row0, CB)])


NS = 4
VC = 3200
VSP = 8 * VC
VP = NS * VSP


def _tr_body(x_ref, o_ref):
    k = pl.program_id(1)
    eye = jnp.eye(8 * D, dtype=jnp.float32)

    def tpose(y):
        return jax.lax.dot_general(
            y, eye, (((0,), (0,)), ((), ())),
            preferred_element_type=jnp.float32)

    def dyn(kk):
        rows = []
        for jj in range(8):
            lo = pl.multiple_of((8 * kk + jj) * VC, 128)
            rows.append(x_ref[0, :, pl.ds(lo, VC)])
        return tpose(jnp.concatenate(rows, axis=0))

    def last():
        rows = []
        for jj in range(8):
            lo = (8 * (NS - 1) + jj) * VC
            hi = min(lo + VC, V)
            xj = x_ref[0, :, lo:hi]
            if hi - lo < VC:
                xj = jnp.concatenate(
                    [xj, jnp.zeros((D, VC - (hi - lo)), jnp.float32)], axis=1)
            rows.append(xj)
        return tpose(jnp.concatenate(rows, axis=0))

    o_ref[0, 0] = lax.cond(k == NS - 1, last, lambda: dyn(k))


_transpose = pl.pallas_call(
    _tr_body,
    grid=(F, NS),
    in_specs=[pl.BlockSpec((1, D, V), lambda f, k: (f, 0, 0))],
    out_specs=pl.BlockSpec((1, 1, VC, 8 * D), lambda f, k: (f, k, 0, 0)),
    out_shape=jax.ShapeDtypeStruct((F, NS, VC, 8 * D), jnp.float32),
)


BM = 2048


def _mlp_body(xu_ref, xi_ref, cw0, cb0, cw1, cb1, cw2, cb2,
              vw0, vb0, vw1, vb1, vw2, vb2, ctr_ref, ctcvr_ref):
    xu = xu_ref[...]
    xi = xi_ref[...]

    def tower(w0, b0, w1, b1, w2, b2):
        w0v = w0[...]
        h = (jnp.dot(xu, w0v[:D], preferred_element_type=jnp.float32)
             + jnp.dot(xi, w0v[D:], preferred_element_type=jnp.float32)
             + b0[...])
        h = jnp.maximum(h, 0.0)
        h = jnp.dot(h, w1[...], preferred_element_type=jnp.float32) + b1[...]
        h = jnp.maximum(h, 0.0)
        z = jnp.sum(h * w2[...], axis=1, keepdims=True) + b2[...]
        return jax.nn.sigmoid(z)

    ctr = tower(cw0, cb0, cw1, cb1, cw2, cb2)
    cvr = tower(vw0, vb0, vw1, vb1, vw2, vb2)
    ctr_ref[...] = ctr
    ctcvr_ref[...] = ctr * cvr


def _full(shape):
    return pl.BlockSpec(shape, lambda i: (0, 0))


_mlp = pl.pallas_call(
    _mlp_body,
    grid=(B // BM,),
    in_specs=[
        pl.BlockSpec((BM, D), lambda i: (i, 0)),
        pl.BlockSpec((BM, D), lambda i: (i, 0)),
        _full((2 * D, 256)), _full((1, 256)),
        _full((256, 128)), _full((1, 128)),
        _full((1, 128)), _full((1, 1)),
        _full((2 * D, 256)), _full((1, 256)),
        _full((256, 128)), _full((1, 128)),
        _full((1, 128)), _full((1, 1)),
    ],
    out_specs=[pl.BlockSpec((BM, 1), lambda i: (i, 0)),
               pl.BlockSpec((BM, 1), lambda i: (i, 0))],
    out_shape=[jax.ShapeDtypeStruct((B, 1), jnp.float32),
               jax.ShapeDtypeStruct((B, 1), jnp.float32)],
)


def kernel(indices, user_table, item_table,
           ctr_W0, ctr_b0, ctr_W1, ctr_b1, ctr_W2, ctr_b2,
           cvr_W0, cvr_b0, cvr_W1, cvr_b1, cvr_W2, cvr_b2):
    idx = indices.astype(jnp.int32)
    j = idx // VC
    ridx = (j // 8) * VSP + (idx % VC) * 8 + (j % 8)
    fidx = (ridx + (jnp.arange(F, dtype=jnp.int32) * VP)[None, :]).reshape(-1)
    uflat = _transpose(user_table.transpose(0, 2, 1)).reshape(F * VP, D)
    pu = _pool(uflat, fidx)
    iflat = _transpose(item_table.transpose(0, 2, 1)).reshape(F * VP, D)
    pi = _pool(iflat, fidx)
    ctr, ctcvr = _mlp(
        pu, pi,
        ctr_W0, ctr_b0.reshape(1, -1), ctr_W1, ctr_b1.reshape(1, -1),
        ctr_W2.reshape(1, -1), ctr_b2.reshape(1, 1),
        cvr_W0, cvr_b0.reshape(1, -1), cvr_W1, cvr_b1.reshape(1, -1),
        cvr_W2.reshape(1, -1), cvr_b2.reshape(1, 1),
    )
    return (ctr, ctcvr)


# speedup vs baseline: 1.4927x; 1.4927x over previous
import functools

import jax
import jax.numpy as jnp
from jax import lax
from jax.experimental import pallas as pl
from jax.experimental.pallas import tpu as pltpu
from jax.experimental.pallas import tpu_sc as plsc

B = 16384
F = 26
V = 100000
D = 16

NC = 2
NSC = 16
NW = NC * NSC
BPW = B // NW
CB = 64
NCHUNK = BPW // CB
RING = 4

_sc_mesh = plsc.VectorSubcoreMesh(core_axis_name="c", subcore_axis_name="s")


@functools.partial(
    pl.kernel,
    mesh=_sc_mesh,
    compiler_params=pltpu.CompilerParams(use_tc_tiling_on_sc=False),
    out_type=jax.ShapeDtypeStruct((B, D), jnp.float32),
    scratch_types=[
        pltpu.VMEM((RING, CB * F), jnp.int32),
        pltpu.VMEM((RING, CB * F, D), jnp.float32),
        pltpu.VMEM((CB, D), jnp.float32),
        pltpu.SemaphoreType.DMA,
        pltpu.SemaphoreType.DMA,
        pltpu.SemaphoreType.DMA,
        pltpu.SemaphoreType.DMA,
    ],
)
def _pool(tflat, fidx, out, idx_v, rows_v, out_v, sem0, sem1, sem2, sem3):
    wid = lax.axis_index("s") * NC + lax.axis_index("c")
    base = wid * BPW
    sems = (sem0, sem1, sem2, sem3)

    def fire(c):
        b = c % RING
        row0 = base + c * CB
        pltpu.sync_copy(fidx.at[pl.ds(row0 * F, CB * F)], idx_v.at[b])
        return pltpu.async_copy(tflat.at[idx_v.at[b]], rows_v.at[b], sems[b])

    inflight = [fire(c) for c in range(RING - 1)]
    for c in range(NCHUNK):
        b = c % RING
        row0 = base + c * CB
        cp = inflight.pop(0)
        if c + RING - 1 < NCHUNK:
            inflight.append(fire(c + RING - 1))
        cp.wait()

        def body(bb, _):
            acc = rows_v[b, bb * F, :]
            for f in range(1, F):
                acc = acc + rows_v[b, bb * F + f, :]
            out_v[bb, :] = acc
            return 0

        lax.fori_loop(0, CB, body, 0)
        pltpu.sync_copy(out_v, out.at[pl.ds(row0, CB)])


NS = 4
VC = 3200
VSP = 8 * VC
VP = NS * VSP


def _tr_body(x_ref, o_ref):
    eye = jnp.eye(8 * D, dtype=jnp.float32)
    for k in range(NS):
        rows = []
        for jj in range(8):
            lo = (8 * k + jj) * VC
            hi = min(lo + VC, V)
            xj = x_ref[0, :, lo:hi]
            if hi - lo < VC:
                xj = jnp.concatenate(
                    [xj, jnp.zeros((D, VC - (hi - lo)), jnp.float32)], axis=1)
            rows.append(xj)
        y = jnp.concatenate(rows, axis=0)
        o_ref[0, k] = jax.lax.dot_general(
            y, eye, (((0,), (0,)), ((), ())),
            preferred_element_type=jnp.float32)


_transpose = pl.pallas_call(
    _tr_body,
    grid=(F,),
    in_specs=[pl.BlockSpec((1, D, V), lambda f: (f, 0, 0))],
    out_specs=pl.BlockSpec((1, NS, VC, 8 * D), lambda f: (f, 0, 0, 0)),
    out_shape=jax.ShapeDtypeStruct((F, NS, VC, 8 * D), jnp.float32),
)


BM = 2048


def _mlp_body(xu_ref, xi_ref, cw0, cb0, cw1, cb1, cw2, cb2,
              vw0, vb0, vw1, vb1, vw2, vb2, ctr_ref, ctcvr_ref):
    xu = xu_ref[...]
    xi = xi_ref[...]

    def tower(w0, b0, w1, b1, w2, b2):
        w0v = w0[...]
        h = (jnp.dot(xu, w0v[:D], preferred_element_type=jnp.float32)
             + jnp.dot(xi, w0v[D:], preferred_element_type=jnp.float32)
             + b0[...])
        h = jnp.maximum(h, 0.0)
        h = jnp.dot(h, w1[...], preferred_element_type=jnp.float32) + b1[...]
        h = jnp.maximum(h, 0.0)
        z = jnp.sum(h * w2[...], axis=1, keepdims=True) + b2[...]
        return jax.nn.sigmoid(z)

    ctr = tower(cw0, cb0, cw1, cb1, cw2, cb2)
    cvr = tower(vw0, vb0, vw1, vb1, vw2, vb2)
    ctr_ref[...] = ctr
    ctcvr_ref[...] = ctr * cvr


def _full(shape):
    return pl.BlockSpec(shape, lambda i: (0, 0))


_mlp = pl.pallas_call(
    _mlp_body,
    grid=(B // BM,),
    in_specs=[
        pl.BlockSpec((BM, D), lambda i: (i, 0)),
        pl.BlockSpec((BM, D), lambda i: (i, 0)),
        _full((2 * D, 256)), _full((1, 256)),
        _full((256, 128)), _full((1, 128)),
        _full((1, 128)), _full((1, 1)),
        _full((2 * D, 256)), _full((1, 256)),
        _full((256, 128)), _full((1, 128)),
        _full((1, 128)), _full((1, 1)),
    ],
    out_specs=[pl.BlockSpec((BM, 1), lambda i: (i, 0)),
               pl.BlockSpec((BM, 1), lambda i: (i, 0))],
    out_shape=[jax.ShapeDtypeStruct((B, 1), jnp.float32),
               jax.ShapeDtypeStruct((B, 1), jnp.float32)],
)


def kernel(indices, user_table, item_table,
           ctr_W0, ctr_b0, ctr_W1, ctr_b1, ctr_W2, ctr_b2,
           cvr_W0, cvr_b0, cvr_W1, cvr_b1, cvr_W2, cvr_b2):
    idx = indices.astype(jnp.int32)
    j = idx // VC
    ridx = (j // 8) * VSP + (idx % VC) * 8 + (j % 8)
    fidx = (ridx + (jnp.arange(F, dtype=jnp.int32) * VP)[None, :]).reshape(-1)
    uflat = _transpose(user_table.transpose(0, 2, 1)).reshape(F * VP, D)
    pu = _pool(uflat, fidx)
    iflat = _transpose(item_table.transpose(0, 2, 1)).reshape(F * VP, D)
    pi = _pool(iflat, fidx)
    ctr, ctcvr = _mlp(
        pu, pi,
        ctr_W0, ctr_b0.reshape(1, -1), ctr_W1, ctr_b1.reshape(1, -1),
        ctr_W2.reshape(1, -1), ctr_b2.reshape(1, 1),
        cvr_W0, cvr_b0.reshape(1, -1), cvr_W1, cvr_b1.reshape(1, -1),
        cvr_W2.reshape(1, -1), cvr_b2.reshape(1, 1),
    )
    return (ctr, ctcvr)
